# Initial kernel scaffold; baseline (speedup 1.0000x reference)
import jax
import jax.numpy as jnp
from jax.experimental import pallas as pl


def kernel(mesh_pos, mesh_norm, mesh_x, mesh_edge_index, mesh_batch, conn_x, conn_adj, params):
    raise NotImplementedError("write your pallas kernel here")


# speedup vs baseline: 4.6359x; 4.6359x over previous
import functools

import jax
import jax.numpy as jnp
from jax import lax
from jax.experimental import pallas as pl
from jax.experimental.pallas import tpu as pltpu
from jax.experimental.pallas import tpu_sc as plsc

N = 50000
E = 800000
B = 16
HID = 64
NCONN = 87

HALF = N // 2
TILES = 16
CHUNK = 128
GROUP = 8
NBUF = 3
CPT = 400
NGROUP = CPT // GROUP
EPAD = TILES * CPT * CHUNK
IDXROWS = EPAD // CHUNK
ZROWS = 1600
ACC = TILES * ZROWS
DUMMY = HALF

RB = 2000
GRID = N // RB

_prec = lax.Precision.HIGHEST

_sc_mesh = plsc.VectorSubcoreMesh(core_axis_name="c", subcore_axis_name="s")
_sc_params = pltpu.CompilerParams(use_tc_tiling_on_sc=False)


@functools.partial(
    pl.kernel,
    out_type=jax.ShapeDtypeStruct((2 * ACC, 16), jnp.float32),
    mesh=_sc_mesh,
    scratch_types=[
        pltpu.VMEM_SHARED((ACC, 16), jnp.float32),
        pltpu.VMEM((GROUP, CHUNK), jnp.int32),
        pltpu.VMEM((CHUNK, 16), jnp.float32),
        pltpu.SemaphoreType.DMA,
    ],
    compiler_params=_sc_params,
)
def _sc_degree(ldst_hbm, zeros16_hbm, ones16_hbm, out_hbm, acc, didx, ones_v,
               sem):
    c = lax.axis_index("c")
    s = lax.axis_index("s")
    pltpu.sync_copy(zeros16_hbm, acc.at[pl.ds(s * ZROWS, ZROWS)])
    pltpu.sync_copy(ones16_hbm, ones_v)
    plsc.subcore_barrier()

    def body(g, carry):
        row0 = s * CPT + g * GROUP
        pltpu.sync_copy(ldst_hbm.at[c, pl.ds(row0, GROUP)], didx)
        handles = [
            pltpu.async_copy(ones_v, acc.at[didx.at[b]], sem, add=True)
            for b in range(GROUP)
        ]
        for h in handles:
            h.wait()
        return carry

    lax.fori_loop(0, NGROUP, body, 0)
    plsc.subcore_barrier()
    pltpu.sync_copy(acc.at[pl.ds(s * ZROWS, ZROWS)],
                    out_hbm.at[pl.ds(c * ACC + s * ZROWS, ZROWS)])


@functools.partial(
    pl.kernel,
    out_type=jax.ShapeDtypeStruct((2 * ACC, HID), jnp.float32),
    mesh=_sc_mesh,
    scratch_types=[
        pltpu.VMEM_SHARED((ACC, HID), jnp.float32),
        pltpu.VMEM((GROUP, CHUNK), jnp.int32),
        pltpu.VMEM((GROUP, CHUNK), jnp.int32),
        pltpu.VMEM((NBUF, CHUNK, HID), jnp.float32),
        pltpu.SemaphoreType.DMA,
        pltpu.SemaphoreType.DMA,
    ],
    compiler_params=_sc_params,
)
def _sc_scatter(y_hbm, src_hbm, ldst_hbm, zeros_hbm, out_hbm,
                acc, sidx, didx, rows, gsem, ssem):
    c = lax.axis_index("c")
    s = lax.axis_index("s")
    pltpu.sync_copy(zeros_hbm, acc.at[pl.ds(s * ZROWS, ZROWS)])
    plsc.subcore_barrier()

    def gather(b):
        return pltpu.async_copy(y_hbm.at[sidx.at[b]], rows.at[b % NBUF], gsem)

    def scatter(b):
        return pltpu.async_copy(rows.at[b % NBUF], acc.at[didx.at[b]], ssem,
                                add=True)

    def body(g, carry):
        row0 = s * CPT + g * GROUP
        pltpu.sync_copy(src_hbm.at[pl.ds(row0, GROUP)], sidx)
        pltpu.sync_copy(ldst_hbm.at[c, pl.ds(row0, GROUP)], didx)
        gh = {0: gather(0), 1: gather(1)}
        sh = {}
        for b in range(GROUP):
            gh[b].wait()
            sh[b] = scatter(b)
            nb = b + 2
            if nb < GROUP:
                if b >= 1:
                    sh[b - 1].wait()
                gh[nb] = gather(nb)
        for t in range(GROUP - NBUF, GROUP):
            sh[t].wait()
        return carry

    lax.fori_loop(0, NGROUP, body, 0)
    plsc.subcore_barrier()
    pltpu.sync_copy(acc.at[pl.ds(s * ZROWS, ZROWS)],
                    out_hbm.at[pl.ds(c * ACC + s * ZROWS, ZROWS)])


def _stats_body(x_ref, sums_ref, sq_ref):
    i = pl.program_id(0)

    @pl.when(i == 0)
    def _():
        sums_ref[...] = jnp.zeros_like(sums_ref)
        sq_ref[...] = jnp.zeros_like(sq_ref)

    x = x_ref[...]
    sums_ref[...] += jnp.sum(x, axis=0, keepdims=True)
    sq_ref[...] += jnp.sum(x * x, axis=0, keepdims=True)


def _stats(x):
    c = x.shape[1]
    return pl.pallas_call(
        _stats_body,
        grid=(GRID,),
        in_specs=[pl.BlockSpec((RB, c), lambda i: (i, 0))],
        out_specs=[pl.BlockSpec((1, c), lambda i: (0, 0)),
                   pl.BlockSpec((1, c), lambda i: (0, 0))],
        out_shape=[jax.ShapeDtypeStruct((1, c), jnp.float32),
                   jax.ShapeDtypeStruct((1, c), jnp.float32)],
    )(x)


def _nms_body(x_ref, sums_ref, sq_ref, g_ref, b_ref, w_ref, deg_ref, y_ref):
    mu = sums_ref[...] / N
    var = sq_ref[...] / N - mu * mu
    rstd = lax.rsqrt(var + 1e-5)
    h = (x_ref[...] - mu) * (rstd * g_ref[...]) + b_ref[...]
    y = lax.dot_general(h, w_ref[...], (((1,), (0,)), ((), ())),
                        precision=_prec, preferred_element_type=jnp.float32)
    dinv = lax.rsqrt(deg_ref[:, 0:1] + 1.0)
    y_ref[...] = y * dinv


def _nms(x, sums, sq, g, b, w, deg):
    c = x.shape[1]
    return pl.pallas_call(
        _nms_body,
        grid=(GRID,),
        in_specs=[pl.BlockSpec((RB, c), lambda i: (i, 0)),
                  pl.BlockSpec((1, c), lambda i: (0, 0)),
                  pl.BlockSpec((1, c), lambda i: (0, 0)),
                  pl.BlockSpec((1, c), lambda i: (0, 0)),
                  pl.BlockSpec((1, c), lambda i: (0, 0)),
                  pl.BlockSpec((c, HID), lambda i: (0, 0)),
                  pl.BlockSpec((RB, 16), lambda i: (i, 0))],
        out_specs=pl.BlockSpec((RB, HID), lambda i: (i, 0)),
        out_shape=jax.ShapeDtypeStruct((N, HID), jnp.float32),
    )(x, sums, sq, g, b, w, deg)


def _post_body(s_ref, y_ref, deg_ref, b_ref, r_ref, sums_ref, sq_ref):
    i = pl.program_id(0)

    @pl.when(i == 0)
    def _():
        sums_ref[...] = jnp.zeros_like(sums_ref)
        sq_ref[...] = jnp.zeros_like(sq_ref)

    dinv = lax.rsqrt(deg_ref[:, 0:1] + 1.0)
    g = dinv * (s_ref[...] + y_ref[...]) + b_ref[...]
    r = jnp.maximum(g, 0.0)
    r_ref[...] = r
    sums_ref[...] += jnp.sum(r, axis=0, keepdims=True)
    sq_ref[...] += jnp.sum(r * r, axis=0, keepdims=True)


def _post(s, y, deg, b):
    return pl.pallas_call(
        _post_body,
        grid=(GRID,),
        in_specs=[pl.BlockSpec((RB, HID), lambda i: (i, 0)),
                  pl.BlockSpec((RB, HID), lambda i: (i, 0)),
                  pl.BlockSpec((RB, 16), lambda i: (i, 0)),
                  pl.BlockSpec((1, HID), lambda i: (0, 0))],
        out_specs=[pl.BlockSpec((RB, HID), lambda i: (i, 0)),
                   pl.BlockSpec((1, HID), lambda i: (0, 0)),
                   pl.BlockSpec((1, HID), lambda i: (0, 0))],
        out_shape=[jax.ShapeDtypeStruct((N, HID), jnp.float32),
                   jax.ShapeDtypeStruct((1, HID), jnp.float32),
                   jax.ShapeDtypeStruct((1, HID), jnp.float32)],
    )(s, y, deg, b)


def _pool_body(r_ref, sums_ref, sq_ref, g_ref, b_ref, batch_ref,
               ms_ref, mc_ref):
    i = pl.program_id(0)

    @pl.when(i == 0)
    def _():
        ms_ref[...] = jnp.zeros_like(ms_ref)
        mc_ref[...] = jnp.zeros_like(mc_ref)

    mu = sums_ref[...] / N
    var = sq_ref[...] / N - mu * mu
    rstd = lax.rsqrt(var + 1e-5)
    h = (r_ref[...] - mu) * (rstd * g_ref[...]) + b_ref[...]
    ids = lax.broadcasted_iota(jnp.int32, (1, B), 1)
    oh = (batch_ref[...] == ids).astype(jnp.float32)
    ms_ref[...] += lax.dot_general(oh, h, (((0,), (0,)), ((), ())),
                                   precision=_prec,
                                   preferred_element_type=jnp.float32)
    mc_ref[...] += jnp.sum(oh, axis=0, keepdims=True)


def _pool(r, sums, sq, g, b, batch2d):
    return pl.pallas_call(
        _pool_body,
        grid=(GRID,),
        in_specs=[pl.BlockSpec((RB, HID), lambda i: (i, 0)),
                  pl.BlockSpec((1, HID), lambda i: (0, 0)),
                  pl.BlockSpec((1, HID), lambda i: (0, 0)),
                  pl.BlockSpec((1, HID), lambda i: (0, 0)),
                  pl.BlockSpec((1, HID), lambda i: (0, 0)),
                  pl.BlockSpec((RB, 1), lambda i: (i, 0))],
        out_specs=[pl.BlockSpec((B, HID), lambda i: (0, 0)),
                   pl.BlockSpec((1, B), lambda i: (0, 0))],
        out_shape=[jax.ShapeDtypeStruct((B, HID), jnp.float32),
                   jax.ShapeDtypeStruct((1, B), jnp.float32)],
    )(r, sums, sq, g, b, batch2d)


def _conn_body(adj_ref, x_ref,
               rw1, rb1, ro1, pr1, rw2, rb2, ro2, pr2, rw3, rb3, ro3, pr3,
               out_ref):
    adjm = adj_ref[0]
    h = x_ref[0]
    for rw, rb, ro, pr in ((rw1, rb1, ro1, pr1),
                           (rw2, rb2, ro2, pr2),
                           (rw3, rb3, ro3, pr3)):
        agg = lax.dot_general(adjm, h, (((1,), (0,)), ((), ())),
                              precision=_prec,
                              preferred_element_type=jnp.float32)
        h2 = (lax.dot_general(agg, rw[...], (((1,), (0,)), ((), ())),
                              precision=_prec,
                              preferred_element_type=jnp.float32)
              + rb[...]
              + lax.dot_general(h, ro[...], (((1,), (0,)), ((), ())),
                                precision=_prec,
                                preferred_element_type=jnp.float32))
        a = pr[0, 0]
        h = jnp.where(h2 >= 0, h2, a * h2)
    out_ref[0] = h


def _conn(adj, x, ws):
    full = lambda shape: pl.BlockSpec(shape, lambda bi: tuple(0 for _ in shape))
    in_specs = [pl.BlockSpec((1, NCONN, NCONN), lambda bi: (bi, 0, 0)),
                pl.BlockSpec((1, NCONN, NCONN), lambda bi: (bi, 0, 0))]
    for w in ws:
        in_specs.append(full(w.shape))
    return pl.pallas_call(
        _conn_body,
        grid=(B,),
        in_specs=in_specs,
        out_specs=pl.BlockSpec((1, NCONN, HID), lambda bi: (bi, 0, 0)),
        out_shape=jax.ShapeDtypeStruct((B, NCONN, HID), jnp.float32),
    )(adj, x, *ws)


def _head_body(ms_ref, mc_ref, cf_ref, w1a_ref, w1b_ref, b1_ref, prf_ref,
               w2_ref, b2_ref, out_ref):
    cnt = jnp.maximum(mc_ref[...], 1.0)
    mf = ms_ref[...] / jnp.transpose(cnt)
    y1 = (lax.dot_general(mf, w1a_ref[...], (((1,), (0,)), ((), ())),
                          precision=_prec, preferred_element_type=jnp.float32)
          + lax.dot_general(cf_ref[...], w1b_ref[...], (((1,), (0,)), ((), ())),
                            precision=_prec,
                            preferred_element_type=jnp.float32)
          + b1_ref[...])
    a = prf_ref[0, 0]
    y1 = jnp.where(y1 >= 0, y1, a * y1)
    out_ref[...] = lax.dot_general(y1, w2_ref[...], (((1,), (0,)), ((), ())),
                                   precision=_prec,
                                   preferred_element_type=jnp.float32) + b2_ref[...]


def _head(ms, mc, cf, w1a, w1b, b1, prf, w2, b2):
    return pl.pallas_call(
        _head_body,
        out_shape=jax.ShapeDtypeStruct((B, 1), jnp.float32),
    )(ms, mc, cf, w1a, w1b, b1, prf, w2, b2)


def kernel(mesh_pos, mesh_norm, mesh_x, mesh_edge_index, mesh_batch,
           conn_x, conn_adj, params):
    p = params
    f32 = jnp.float32

    src = mesh_edge_index[0].astype(jnp.int32)
    dst = mesh_edge_index[1].astype(jnp.int32)

    pad = EPAD - E
    srcp = jnp.concatenate([src, jnp.zeros((pad,), jnp.int32)])
    srcp = srcp.reshape(IDXROWS, CHUNK)
    dstp = jnp.concatenate([dst, jnp.full((pad,), -1, jnp.int32)])
    ld0 = jnp.where((dstp >= 0) & (dstp < HALF), dstp, DUMMY)
    ld1 = jnp.where(dstp >= HALF, dstp - HALF, DUMMY)
    ldst = jnp.stack([ld0, ld1]).reshape(2, IDXROWS, CHUNK)

    zeros16 = jnp.zeros((ZROWS, 16), f32)
    ones16 = jnp.ones((CHUNK, 16), f32)
    degp = _sc_degree(ldst, zeros16, ones16)
    deg = degp.reshape(2, ACC, 16)[:, :HALF, :].reshape(N, 16)

    x0 = jnp.concatenate([mesh_pos, mesh_norm, mesh_x], axis=1)
    sums, sq = _stats(x0)

    zerosz = jnp.zeros((ZROWS, HID), f32)
    h_pre = x0
    bn_g, bn_b = p["bn0_g"], p["bn0_b"]
    for i in range(1, 5):
        c = h_pre.shape[1]
        y = _nms(h_pre, sums, sq, bn_g.reshape(1, c), bn_b.reshape(1, c),
                 p["gcnW%d" % i], deg)
        sp = _sc_scatter(y, srcp, ldst, zerosz)
        sv = sp.reshape(2, ACC, HID)[:, :HALF].reshape(N, HID)
        h_pre, sums, sq = _post(sv, y, deg, p["gcnb%d" % i].reshape(1, HID))
        bn_g, bn_b = p["bn%d_g" % i], p["bn%d_b" % i]

    batch2d = mesh_batch.astype(jnp.int32).reshape(N, 1)
    ms, mc = _pool(h_pre, sums, sq, bn_g.reshape(1, HID),
                   bn_b.reshape(1, HID), batch2d)

    ws = []
    for i in range(1, 4):
        ws += [p["relW%d" % i], p["relb%d" % i].reshape(1, HID),
               p["rootW%d" % i], p["pr%d" % i].reshape(1, 1)]
    h3 = _conn(conn_adj, conn_x, ws)
    cf = h3.reshape(B, NCONN * HID)

    out = _head(ms, mc, cf,
                p["lin1W"][:HID], p["lin1W"][HID:],
                p["lin1b"].reshape(1, HID), p["prF"].reshape(1, 1),
                p["lin2W"], p["lin2b"].reshape(1, 1))
    return out


# speedup vs baseline: 8.4785x; 1.8289x over previous
import functools

import jax
import jax.numpy as jnp
from jax import lax
from jax.experimental import pallas as pl
from jax.experimental.pallas import tpu as pltpu
from jax.experimental.pallas import tpu_sc as plsc

N = 50000
E = 800000
B = 16
HID = 64
NCONN = 87

HALF = N // 2
TILES = 16
CHUNK = 128
GROUP = 16
NBUF = 5
CPT = 400
NGROUP = CPT // GROUP
EPAD = TILES * CPT * CHUNK
IDXROWS = EPAD // CHUNK
ZROWS = 1600
ACC = TILES * ZROWS
DUMMY = HALF
HW = HID // 2
SZROWS = 3128
SACC = TILES * SZROWS
SDUMMY = N

RB = 2000
GRID = N // RB

_prec = lax.Precision.DEFAULT

_sc_mesh = plsc.VectorSubcoreMesh(core_axis_name="c", subcore_axis_name="s")
_sc_params = pltpu.CompilerParams(use_tc_tiling_on_sc=False)


@functools.partial(
    pl.kernel,
    out_type=jax.ShapeDtypeStruct((2 * ACC, 16), jnp.float32),
    mesh=_sc_mesh,
    scratch_types=[
        pltpu.VMEM_SHARED((ACC, 16), jnp.float32),
        pltpu.VMEM((GROUP, CHUNK), jnp.int32),
        pltpu.VMEM((CHUNK, 16), jnp.float32),
        pltpu.SemaphoreType.DMA,
    ],
    compiler_params=_sc_params,
)
def _sc_degree(ldst_hbm, zeros16_hbm, ones16_hbm, out_hbm, acc, didx, ones_v,
               sem):
    c = lax.axis_index("c")
    s = lax.axis_index("s")
    pltpu.sync_copy(zeros16_hbm, acc.at[pl.ds(s * ZROWS, ZROWS)])
    pltpu.sync_copy(ones16_hbm, ones_v)
    plsc.subcore_barrier()

    def body(g, carry):
        row0 = s * CPT + g * GROUP
        pltpu.sync_copy(ldst_hbm.at[c, pl.ds(row0, GROUP)], didx)
        handles = [
            pltpu.async_copy(ones_v, acc.at[didx.at[b]], sem, add=True)
            for b in range(GROUP)
        ]
        for h in handles:
            h.wait()
        return carry

    lax.fori_loop(0, NGROUP, body, 0)
    plsc.subcore_barrier()
    pltpu.sync_copy(acc.at[pl.ds(s * ZROWS, ZROWS)],
                    out_hbm.at[pl.ds(c * ACC + s * ZROWS, ZROWS)])


@functools.partial(
    pl.kernel,
    out_type=jax.ShapeDtypeStruct((2, SACC, HW), jnp.float32),
    mesh=_sc_mesh,
    scratch_types=[
        pltpu.VMEM_SHARED((SACC, HW), jnp.float32),
        pltpu.VMEM((GROUP, CHUNK), jnp.int32),
        pltpu.VMEM((GROUP, CHUNK), jnp.int32),
        pltpu.VMEM((NBUF, CHUNK, HW), jnp.float32),
        pltpu.SemaphoreType.DMA,
        pltpu.SemaphoreType.DMA,
    ],
    compiler_params=_sc_params,
)
def _sc_scatter(y_hbm, src_hbm, didx_hbm, zeros_hbm, out_hbm,
                acc, sidx, didx, rows, gsem, ssem):
    c = lax.axis_index("c")
    s = lax.axis_index("s")
    pltpu.sync_copy(zeros_hbm, acc.at[pl.ds(s * SZROWS, SZROWS)])
    plsc.subcore_barrier()

    def gather(b):
        return pltpu.async_copy(y_hbm.at[c].at[sidx.at[b]], rows.at[b % NBUF],
                                gsem)

    def scatter(b):
        return pltpu.async_copy(rows.at[b % NBUF], acc.at[didx.at[b]], ssem,
                                add=True)

    lead = NBUF - 1

    def body(g, carry):
        row0 = s * CPT + g * GROUP
        pltpu.sync_copy(src_hbm.at[pl.ds(row0, GROUP)], sidx)
        pltpu.sync_copy(didx_hbm.at[pl.ds(row0, GROUP)], didx)
        gh = {b: gather(b) for b in range(lead)}
        sh = {}
        for b in range(GROUP):
            gh[b].wait()
            sh[b] = scatter(b)
            nb = b + lead
            if nb < GROUP:
                if b >= 1:
                    sh[b - 1].wait()
                gh[nb] = gather(nb)
        for t in range(max(GROUP - lead - 1, 0), GROUP):
            sh[t].wait()
        return carry

    lax.fori_loop(0, NGROUP, body, 0)
    plsc.subcore_barrier()
    pltpu.sync_copy(acc.at[pl.ds(s * SZROWS, SZROWS)],
                    out_hbm.at[c, pl.ds(s * SZROWS, SZROWS)])


def _acc_stats(i, x, sums_ref, csq_ref, bmu_ref):
    @pl.when(i == 0)
    def _():
        sums_ref[...] = jnp.zeros_like(sums_ref)
        csq_ref[...] = jnp.zeros_like(csq_ref)

    s = jnp.sum(x, axis=0, keepdims=True)
    mu = s / RB
    d = x - mu
    sums_ref[...] += s
    csq_ref[...] += jnp.sum(d * d, axis=0, keepdims=True)
    bmu_ref[0] = mu


def _var_from_stats(sums, csq, bmu):
    mu = sums / N
    d = bmu.reshape(GRID, -1) - mu
    var = csq / N + jnp.sum(d * d, axis=0, keepdims=True) * (RB / N)
    return mu, lax.rsqrt(var + 1e-5)


def _stats_body(x_ref, sums_ref, csq_ref, bmu_ref):
    _acc_stats(pl.program_id(0), x_ref[...], sums_ref, csq_ref, bmu_ref)


def _stats(x):
    c = x.shape[1]
    return pl.pallas_call(
        _stats_body,
        grid=(GRID,),
        in_specs=[pl.BlockSpec((RB, c), lambda i: (i, 0))],
        out_specs=[pl.BlockSpec((1, c), lambda i: (0, 0)),
                   pl.BlockSpec((1, c), lambda i: (0, 0)),
                   pl.BlockSpec((1, 1, c), lambda i: (i, 0, 0))],
        out_shape=[jax.ShapeDtypeStruct((1, c), jnp.float32),
                   jax.ShapeDtypeStruct((1, c), jnp.float32),
                   jax.ShapeDtypeStruct((GRID, 1, c), jnp.float32)],
    )(x)


def _nms_body(x_ref, sums_ref, csq_ref, bmu_ref, g_ref, b_ref, w_ref,
              deg_ref, y_ref):
    mu, rstd = _var_from_stats(sums_ref[...], csq_ref[...], bmu_ref[...])
    h = (x_ref[...] - mu) * (rstd * g_ref[...]) + b_ref[...]
    y = lax.dot_general(h, w_ref[...], (((1,), (0,)), ((), ())),
                        precision=_prec, preferred_element_type=jnp.float32)
    dinv = lax.rsqrt(deg_ref[:, 0:1] + 1.0)
    y = y * dinv
    y_ref[0] = y[:, :HW]
    y_ref[1] = y[:, HW:]


def _nms(x, sums, csq, bmu, g, b, w, deg):
    c = x.shape[1]
    return pl.pallas_call(
        _nms_body,
        grid=(GRID,),
        in_specs=[pl.BlockSpec((RB, c), lambda i: (i, 0)),
                  pl.BlockSpec((1, c), lambda i: (0, 0)),
                  pl.BlockSpec((1, c), lambda i: (0, 0)),
                  pl.BlockSpec((GRID, 1, c), lambda i: (0, 0, 0)),
                  pl.BlockSpec((1, c), lambda i: (0, 0)),
                  pl.BlockSpec((1, c), lambda i: (0, 0)),
                  pl.BlockSpec((c, HID), lambda i: (0, 0)),
                  pl.BlockSpec((RB, 16), lambda i: (i, 0))],
        out_specs=pl.BlockSpec((2, RB, HW), lambda i: (0, i, 0)),
        out_shape=jax.ShapeDtypeStruct((2, N, HW), jnp.float32),
    )(x, sums, csq, bmu, g, b, w, deg)


def _post_body(s_ref, y_ref, deg_ref, b_ref, r_ref, sums_ref, csq_ref,
               bmu_ref):
    dinv = lax.rsqrt(deg_ref[:, 0:1] + 1.0)
    sv = jnp.concatenate([s_ref[0], s_ref[1]], axis=1)
    yv = jnp.concatenate([y_ref[0], y_ref[1]], axis=1)
    g = dinv * (sv + yv) + b_ref[...]
    r = jnp.maximum(g, 0.0)
    r_ref[...] = r
    _acc_stats(pl.program_id(0), r, sums_ref, csq_ref, bmu_ref)


def _post(s, y, deg, b):
    return pl.pallas_call(
        _post_body,
        grid=(GRID,),
        in_specs=[pl.BlockSpec((2, RB, HW), lambda i: (0, i, 0)),
                  pl.BlockSpec((2, RB, HW), lambda i: (0, i, 0)),
                  pl.BlockSpec((RB, 16), lambda i: (i, 0)),
                  pl.BlockSpec((1, HID), lambda i: (0, 0))],
        out_specs=[pl.BlockSpec((RB, HID), lambda i: (i, 0)),
                   pl.BlockSpec((1, HID), lambda i: (0, 0)),
                   pl.BlockSpec((1, HID), lambda i: (0, 0)),
                   pl.BlockSpec((1, 1, HID), lambda i: (i, 0, 0))],
        out_shape=[jax.ShapeDtypeStruct((N, HID), jnp.float32),
                   jax.ShapeDtypeStruct((1, HID), jnp.float32),
                   jax.ShapeDtypeStruct((1, HID), jnp.float32),
                   jax.ShapeDtypeStruct((GRID, 1, HID), jnp.float32)],
    )(s, y, deg, b)


def _pool_body(r_ref, sums_ref, csq_ref, bmu_ref, g_ref, b_ref, batch_ref,
               ms_ref, mc_ref):
    i = pl.program_id(0)

    @pl.when(i == 0)
    def _():
        ms_ref[...] = jnp.zeros_like(ms_ref)
        mc_ref[...] = jnp.zeros_like(mc_ref)

    mu, rstd = _var_from_stats(sums_ref[...], csq_ref[...], bmu_ref[...])
    h = (r_ref[...] - mu) * (rstd * g_ref[...]) + b_ref[...]
    ids = lax.broadcasted_iota(jnp.int32, (1, B), 1)
    oh = (batch_ref[...] == ids).astype(jnp.float32)
    ms_ref[...] += lax.dot_general(oh, h, (((0,), (0,)), ((), ())),
                                   precision=_prec,
                                   preferred_element_type=jnp.float32)
    mc_ref[...] += jnp.sum(oh, axis=0, keepdims=True)


def _pool(r, sums, csq, bmu, g, b, batch2d):
    return pl.pallas_call(
        _pool_body,
        grid=(GRID,),
        in_specs=[pl.BlockSpec((RB, HID), lambda i: (i, 0)),
                  pl.BlockSpec((1, HID), lambda i: (0, 0)),
                  pl.BlockSpec((1, HID), lambda i: (0, 0)),
                  pl.BlockSpec((GRID, 1, HID), lambda i: (0, 0, 0)),
                  pl.BlockSpec((1, HID), lambda i: (0, 0)),
                  pl.BlockSpec((1, HID), lambda i: (0, 0)),
                  pl.BlockSpec((RB, 1), lambda i: (i, 0))],
        out_specs=[pl.BlockSpec((B, HID), lambda i: (0, 0)),
                   pl.BlockSpec((1, B), lambda i: (0, 0))],
        out_shape=[jax.ShapeDtypeStruct((B, HID), jnp.float32),
                   jax.ShapeDtypeStruct((1, B), jnp.float32)],
    )(r, sums, csq, bmu, g, b, batch2d)


def _conn_body(adj_ref, x_ref,
               rw1, rb1, ro1, pr1, rw2, rb2, ro2, pr2, rw3, rb3, ro3, pr3,
               out_ref):
    adjm = adj_ref[0]
    h = x_ref[0]
    for rw, rb, ro, pr in ((rw1, rb1, ro1, pr1),
                           (rw2, rb2, ro2, pr2),
                           (rw3, rb3, ro3, pr3)):
        agg = lax.dot_general(adjm, h, (((1,), (0,)), ((), ())),
                              precision=_prec,
                              preferred_element_type=jnp.float32)
        h2 = (lax.dot_general(agg, rw[...], (((1,), (0,)), ((), ())),
                              precision=_prec,
                              preferred_element_type=jnp.float32)
              + rb[...]
              + lax.dot_general(h, ro[...], (((1,), (0,)), ((), ())),
                                precision=_prec,
                                preferred_element_type=jnp.float32))
        a = pr[0, 0]
        h = jnp.where(h2 >= 0, h2, a * h2)
    out_ref[0] = h


def _conn(adj, x, ws):
    full = lambda shape: pl.BlockSpec(shape, lambda bi: tuple(0 for _ in shape))
    in_specs = [pl.BlockSpec((1, NCONN, NCONN), lambda bi: (bi, 0, 0)),
                pl.BlockSpec((1, NCONN, NCONN), lambda bi: (bi, 0, 0))]
    for w in ws:
        in_specs.append(full(w.shape))
    return pl.pallas_call(
        _conn_body,
        grid=(B,),
        in_specs=in_specs,
        out_specs=pl.BlockSpec((1, NCONN, HID), lambda bi: (bi, 0, 0)),
        out_shape=jax.ShapeDtypeStruct((B, NCONN, HID), jnp.float32),
    )(adj, x, *ws)


def _head_body(ms_ref, mc_ref, cf_ref, w1a_ref, w1b_ref, b1_ref, prf_ref,
               w2_ref, b2_ref, out_ref):
    cnt = jnp.maximum(mc_ref[...], 1.0)
    mf = ms_ref[...] / jnp.transpose(cnt)
    y1 = (lax.dot_general(mf, w1a_ref[...], (((1,), (0,)), ((), ())),
                          precision=_prec, preferred_element_type=jnp.float32)
          + lax.dot_general(cf_ref[...], w1b_ref[...], (((1,), (0,)), ((), ())),
                            precision=_prec,
                            preferred_element_type=jnp.float32)
          + b1_ref[...])
    a = prf_ref[0, 0]
    y1 = jnp.where(y1 >= 0, y1, a * y1)
    out_ref[...] = lax.dot_general(y1, w2_ref[...], (((1,), (0,)), ((), ())),
                                   precision=_prec,
                                   preferred_element_type=jnp.float32) + b2_ref[...]


def _head(ms, mc, cf, w1a, w1b, b1, prf, w2, b2):
    return pl.pallas_call(
        _head_body,
        out_shape=jax.ShapeDtypeStruct((B, 1), jnp.float32),
    )(ms, mc, cf, w1a, w1b, b1, prf, w2, b2)


def kernel(mesh_pos, mesh_norm, mesh_x, mesh_edge_index, mesh_batch,
           conn_x, conn_adj, params):
    p = params
    f32 = jnp.float32

    src = mesh_edge_index[0].astype(jnp.int32)
    dst = mesh_edge_index[1].astype(jnp.int32)

    pad = EPAD - E
    srcp = jnp.concatenate([src, jnp.zeros((pad,), jnp.int32)])
    srcp = srcp.reshape(IDXROWS, CHUNK)
    dstp = jnp.concatenate([dst, jnp.full((pad,), -1, jnp.int32)])
    ld0 = jnp.where((dstp >= 0) & (dstp < HALF), dstp, DUMMY)
    ld1 = jnp.where(dstp >= HALF, dstp - HALF, DUMMY)
    ldst = jnp.stack([ld0, ld1]).reshape(2, IDXROWS, CHUNK)
    didx2 = jnp.where(dstp >= 0, dstp, SDUMMY).reshape(IDXROWS, CHUNK)

    zeros16 = jnp.zeros((ZROWS, 16), f32)
    ones16 = jnp.ones((CHUNK, 16), f32)
    degp = _sc_degree(ldst, zeros16, ones16)
    deg = degp.reshape(2, ACC, 16)[:, :HALF, :].reshape(N, 16)

    x0 = jnp.concatenate([mesh_pos, mesh_norm, mesh_x], axis=1)
    sums, csq, bmu = _stats(x0)

    zerosz = jnp.zeros((SZROWS, HW), f32)
    h_pre = x0
    bn_g, bn_b = p["bn0_g"], p["bn0_b"]
    for i in range(1, 5):
        c = h_pre.shape[1]
        y = _nms(h_pre, sums, csq, bmu, bn_g.reshape(1, c),
                 bn_b.reshape(1, c), p["gcnW%d" % i], deg)
        sp = _sc_scatter(y, srcp, didx2, zerosz)
        sv = sp[:, :N, :]
        h_pre, sums, csq, bmu = _post(sv, y, deg,
                                      p["gcnb%d" % i].reshape(1, HID))
        bn_g, bn_b = p["bn%d_g" % i], p["bn%d_b" % i]

    batch2d = mesh_batch.astype(jnp.int32).reshape(N, 1)
    ms, mc = _pool(h_pre, sums, csq, bmu, bn_g.reshape(1, HID),
                   bn_b.reshape(1, HID), batch2d)

    ws = []
    for i in range(1, 4):
        ws += [p["relW%d" % i], p["relb%d" % i].reshape(1, HID),
               p["rootW%d" % i], p["pr%d" % i].reshape(1, 1)]
    h3 = _conn(conn_adj, conn_x, ws)
    cf = h3.reshape(B, NCONN * HID)

    out = _head(ms, mc, cf,
                p["lin1W"][:HID], p["lin1W"][HID:],
                p["lin1b"].reshape(1, HID), p["prF"].reshape(1, 1),
                p["lin2W"], p["lin2b"].reshape(1, 1))
    return out


# speedup vs baseline: 17.0831x; 2.0149x over previous
import functools

import jax
import jax.numpy as jnp
from jax import lax
from jax.experimental import pallas as pl
from jax.experimental.pallas import tpu as pltpu
from jax.experimental.pallas import tpu_sc as plsc

N = 50000
E = 800000
B = 16
HID = 64
NCONN = 87

HALF = N // 2
TILES = 16
CHUNK = 128
GROUP = 16
NBUF = 5
CPT = 400
NGROUP = CPT // GROUP
EPAD = TILES * CPT * CHUNK
IDXROWS = EPAD // CHUNK
ZROWS = 1600
ACC = TILES * ZROWS
DUMMY = HALF
HW = HID // 2
SZROWS = 3128
SACC = TILES * SZROWS
SDUMMY = N

RB = 2000
GRID = N // RB

_prec = lax.Precision.DEFAULT

_sc_mesh = plsc.VectorSubcoreMesh(core_axis_name="c", subcore_axis_name="s")
_sc_params = pltpu.CompilerParams(use_tc_tiling_on_sc=False)


@functools.partial(
    pl.kernel,
    out_type=jax.ShapeDtypeStruct((2 * ACC, 16), jnp.float32),
    mesh=_sc_mesh,
    scratch_types=[
        pltpu.VMEM_SHARED((ACC, 16), jnp.float32),
        pltpu.VMEM((GROUP, CHUNK), jnp.int32),
        pltpu.VMEM((CHUNK, 16), jnp.float32),
        pltpu.SemaphoreType.DMA,
    ],
    compiler_params=_sc_params,
)
def _sc_degree(ldst_hbm, zeros16_hbm, ones16_hbm, out_hbm, acc, didx, ones_v,
               sem):
    c = lax.axis_index("c")
    s = lax.axis_index("s")
    pltpu.sync_copy(zeros16_hbm, acc.at[pl.ds(s * ZROWS, ZROWS)])
    pltpu.sync_copy(ones16_hbm, ones_v)
    plsc.subcore_barrier()

    def body(g, carry):
        row0 = s * CPT + g * GROUP
        pltpu.sync_copy(ldst_hbm.at[c, pl.ds(row0, GROUP)], didx)
        handles = [
            pltpu.async_copy(ones_v, acc.at[didx.at[b]], sem, add=True)
            for b in range(GROUP)
        ]
        for h in handles:
            h.wait()
        return carry

    lax.fori_loop(0, NGROUP, body, 0)
    plsc.subcore_barrier()
    pltpu.sync_copy(acc.at[pl.ds(s * ZROWS, ZROWS)],
                    out_hbm.at[pl.ds(c * ACC + s * ZROWS, ZROWS)])


@functools.partial(
    pl.kernel,
    out_type=jax.ShapeDtypeStruct((2, SACC, HW), jnp.float32),
    mesh=_sc_mesh,
    scratch_types=[
        pltpu.VMEM_SHARED((SACC, HW), jnp.float32),
        pltpu.VMEM((GROUP, CHUNK), jnp.int32),
        pltpu.VMEM((GROUP, CHUNK), jnp.int32),
        pltpu.VMEM((NBUF, CHUNK, HW), jnp.float32),
        pltpu.SemaphoreType.DMA,
        pltpu.SemaphoreType.DMA,
    ],
    compiler_params=_sc_params,
)
def _sc_scatter(y_hbm, src_hbm, didx_hbm, zeros_hbm, out_hbm,
                acc, sidx, didx, rows, gsem, ssem):
    c = lax.axis_index("c")
    s = lax.axis_index("s")
    pltpu.sync_copy(zeros_hbm, acc.at[pl.ds(s * SZROWS, SZROWS)])
    plsc.subcore_barrier()

    def gather(b):
        return pltpu.async_copy(y_hbm.at[c].at[sidx.at[b]], rows.at[b % NBUF],
                                gsem)

    def scatter(b):
        return pltpu.async_copy(rows.at[b % NBUF], acc.at[didx.at[b]], ssem,
                                add=True)

    lead = NBUF - 1

    def body(g, carry):
        row0 = s * CPT + g * GROUP
        pltpu.sync_copy(src_hbm.at[pl.ds(row0, GROUP)], sidx)
        pltpu.sync_copy(didx_hbm.at[pl.ds(row0, GROUP)], didx)
        gh = {b: gather(b) for b in range(lead)}
        sh = {}
        for b in range(GROUP):
            gh[b].wait()
            sh[b] = scatter(b)
            nb = b + lead
            if nb < GROUP:
                if b >= 1:
                    sh[b - 1].wait()
                gh[nb] = gather(nb)
        for t in range(max(GROUP - lead - 1, 0), GROUP):
            sh[t].wait()
        return carry

    lax.fori_loop(0, NGROUP, body, 0)
    plsc.subcore_barrier()
    pltpu.sync_copy(acc.at[pl.ds(s * SZROWS, SZROWS)],
                    out_hbm.at[c, pl.ds(s * SZROWS, SZROWS)])


def _acc_stats(i, x, sums_ref, csq_ref, bmu_ref):
    @pl.when(i == 0)
    def _():
        sums_ref[...] = jnp.zeros_like(sums_ref)
        csq_ref[...] = jnp.zeros_like(csq_ref)

    s = jnp.sum(x, axis=0, keepdims=True)
    mu = s / RB
    d = x - mu
    sums_ref[...] += s
    csq_ref[...] += jnp.sum(d * d, axis=0, keepdims=True)
    bmu_ref[0] = mu


def _var_from_stats(sums, csq, bmu):
    mu = sums / N
    d = bmu.reshape(GRID, -1) - mu
    var = csq / N + jnp.sum(d * d, axis=0, keepdims=True) * (RB / N)
    return mu, lax.rsqrt(var + 1e-5)


def _stats_body(x_ref, sums_ref, csq_ref, bmu_ref):
    _acc_stats(pl.program_id(0), x_ref[...], sums_ref, csq_ref, bmu_ref)


def _stats(x):
    c = x.shape[1]
    return pl.pallas_call(
        _stats_body,
        grid=(GRID,),
        in_specs=[pl.BlockSpec((RB, c), lambda i: (i, 0))],
        out_specs=[pl.BlockSpec((1, c), lambda i: (0, 0)),
                   pl.BlockSpec((1, c), lambda i: (0, 0)),
                   pl.BlockSpec((1, 1, c), lambda i: (i, 0, 0))],
        out_shape=[jax.ShapeDtypeStruct((1, c), jnp.float32),
                   jax.ShapeDtypeStruct((1, c), jnp.float32),
                   jax.ShapeDtypeStruct((GRID, 1, c), jnp.float32)],
    )(x)


def _nms_body(x_ref, sums_ref, csq_ref, bmu_ref, g_ref, b_ref, w_ref,
              deg_ref, y_ref):
    mu, rstd = _var_from_stats(sums_ref[...], csq_ref[...], bmu_ref[...])
    h = (x_ref[...] - mu) * (rstd * g_ref[...]) + b_ref[...]
    y = lax.dot_general(h, w_ref[...], (((1,), (0,)), ((), ())),
                        precision=_prec, preferred_element_type=jnp.float32)
    dinv = lax.rsqrt(deg_ref[:, 0:1] + 1.0)
    y = y * dinv
    y_ref[0] = y[:, :HW]
    y_ref[1] = y[:, HW:]


def _nms(x, sums, csq, bmu, g, b, w, deg):
    c = x.shape[1]
    return pl.pallas_call(
        _nms_body,
        grid=(GRID,),
        in_specs=[pl.BlockSpec((RB, c), lambda i: (i, 0)),
                  pl.BlockSpec((1, c), lambda i: (0, 0)),
                  pl.BlockSpec((1, c), lambda i: (0, 0)),
                  pl.BlockSpec((GRID, 1, c), lambda i: (0, 0, 0)),
                  pl.BlockSpec((1, c), lambda i: (0, 0)),
                  pl.BlockSpec((1, c), lambda i: (0, 0)),
                  pl.BlockSpec((c, HID), lambda i: (0, 0)),
                  pl.BlockSpec((RB, 16), lambda i: (i, 0))],
        out_specs=pl.BlockSpec((2, RB, HW), lambda i: (0, i, 0)),
        out_shape=jax.ShapeDtypeStruct((2, N, HW), jnp.float32),
    )(x, sums, csq, bmu, g, b, w, deg)


def _post_body(s_ref, y_ref, deg_ref, b_ref, r_ref, sums_ref, csq_ref,
               bmu_ref):
    dinv = lax.rsqrt(deg_ref[:, 0:1] + 1.0)
    sv = jnp.concatenate([s_ref[0], s_ref[1]], axis=1)
    yv = jnp.concatenate([y_ref[0], y_ref[1]], axis=1)
    g = dinv * (sv + yv) + b_ref[...]
    r = jnp.maximum(g, 0.0)
    r_ref[...] = r
    _acc_stats(pl.program_id(0), r, sums_ref, csq_ref, bmu_ref)


def _post(s, y, deg, b):
    return pl.pallas_call(
        _post_body,
        grid=(GRID,),
        in_specs=[pl.BlockSpec((2, RB, HW), lambda i: (0, i, 0)),
                  pl.BlockSpec((2, RB, HW), lambda i: (0, i, 0)),
                  pl.BlockSpec((RB, 16), lambda i: (i, 0)),
                  pl.BlockSpec((1, HID), lambda i: (0, 0))],
        out_specs=[pl.BlockSpec((RB, HID), lambda i: (i, 0)),
                   pl.BlockSpec((1, HID), lambda i: (0, 0)),
                   pl.BlockSpec((1, HID), lambda i: (0, 0)),
                   pl.BlockSpec((1, 1, HID), lambda i: (i, 0, 0))],
        out_shape=[jax.ShapeDtypeStruct((N, HID), jnp.float32),
                   jax.ShapeDtypeStruct((1, HID), jnp.float32),
                   jax.ShapeDtypeStruct((1, HID), jnp.float32),
                   jax.ShapeDtypeStruct((GRID, 1, HID), jnp.float32)],
    )(s, y, deg, b)


def _pool_body(r_ref, sums_ref, csq_ref, bmu_ref, g_ref, b_ref, batch_ref,
               ms_ref, mc_ref):
    i = pl.program_id(0)

    @pl.when(i == 0)
    def _():
        ms_ref[...] = jnp.zeros_like(ms_ref)
        mc_ref[...] = jnp.zeros_like(mc_ref)

    mu, rstd = _var_from_stats(sums_ref[...], csq_ref[...], bmu_ref[...])
    h = (r_ref[...] - mu) * (rstd * g_ref[...]) + b_ref[...]
    ids = lax.broadcasted_iota(jnp.int32, (1, B), 1)
    oh = (batch_ref[...] == ids).astype(jnp.float32)
    ms_ref[...] += lax.dot_general(oh, h, (((0,), (0,)), ((), ())),
                                   precision=_prec,
                                   preferred_element_type=jnp.float32)
    mc_ref[...] += jnp.sum(oh, axis=0, keepdims=True)


def _pool(r, sums, csq, bmu, g, b, batch2d):
    return pl.pallas_call(
        _pool_body,
        grid=(GRID,),
        in_specs=[pl.BlockSpec((RB, HID), lambda i: (i, 0)),
                  pl.BlockSpec((1, HID), lambda i: (0, 0)),
                  pl.BlockSpec((1, HID), lambda i: (0, 0)),
                  pl.BlockSpec((GRID, 1, HID), lambda i: (0, 0, 0)),
                  pl.BlockSpec((1, HID), lambda i: (0, 0)),
                  pl.BlockSpec((1, HID), lambda i: (0, 0)),
                  pl.BlockSpec((RB, 1), lambda i: (i, 0))],
        out_specs=[pl.BlockSpec((B, HID), lambda i: (0, 0)),
                   pl.BlockSpec((1, B), lambda i: (0, 0))],
        out_shape=[jax.ShapeDtypeStruct((B, HID), jnp.float32),
                   jax.ShapeDtypeStruct((1, B), jnp.float32)],
    )(r, sums, csq, bmu, g, b, batch2d)


def _conn_body(adj_ref, x_ref,
               rw1, rb1, ro1, pr1, rw2, rb2, ro2, pr2, rw3, rb3, ro3, pr3,
               out_ref):
    adjm = adj_ref[0]
    h = x_ref[0]
    for rw, rb, ro, pr in ((rw1, rb1, ro1, pr1),
                           (rw2, rb2, ro2, pr2),
                           (rw3, rb3, ro3, pr3)):
        agg = lax.dot_general(adjm, h, (((1,), (0,)), ((), ())),
                              precision=_prec,
                              preferred_element_type=jnp.float32)
        h2 = (lax.dot_general(agg, rw[...], (((1,), (0,)), ((), ())),
                              precision=_prec,
                              preferred_element_type=jnp.float32)
              + rb[...]
              + lax.dot_general(h, ro[...], (((1,), (0,)), ((), ())),
                                precision=_prec,
                                preferred_element_type=jnp.float32))
        a = pr[0, 0]
        h = jnp.where(h2 >= 0, h2, a * h2)
    out_ref[0] = h


def _conn(adj, x, ws):
    full = lambda shape: pl.BlockSpec(shape, lambda bi: tuple(0 for _ in shape))
    in_specs = [pl.BlockSpec((1, NCONN, NCONN), lambda bi: (bi, 0, 0)),
                pl.BlockSpec((1, NCONN, NCONN), lambda bi: (bi, 0, 0))]
    for w in ws:
        in_specs.append(full(w.shape))
    return pl.pallas_call(
        _conn_body,
        grid=(B,),
        in_specs=in_specs,
        out_specs=pl.BlockSpec((1, NCONN, HID), lambda bi: (bi, 0, 0)),
        out_shape=jax.ShapeDtypeStruct((B, NCONN, HID), jnp.float32),
    )(adj, x, *ws)


def _head_body(ms_ref, mc_ref, cf_ref, w1a_ref, w1b_ref, b1_ref, prf_ref,
               w2_ref, b2_ref, out_ref):
    cnt = jnp.maximum(mc_ref[...], 1.0)
    mf = ms_ref[...] / jnp.transpose(cnt)
    y1 = (lax.dot_general(mf, w1a_ref[...], (((1,), (0,)), ((), ())),
                          precision=_prec, preferred_element_type=jnp.float32)
          + lax.dot_general(cf_ref[...], w1b_ref[...], (((1,), (0,)), ((), ())),
                            precision=_prec,
                            preferred_element_type=jnp.float32)
          + b1_ref[...])
    a = prf_ref[0, 0]
    y1 = jnp.where(y1 >= 0, y1, a * y1)
    out_ref[...] = lax.dot_general(y1, w2_ref[...], (((1,), (0,)), ((), ())),
                                   precision=_prec,
                                   preferred_element_type=jnp.float32) + b2_ref[...]


def _head(ms, mc, cf, w1a, w1b, b1, prf, w2, b2):
    return pl.pallas_call(
        _head_body,
        out_shape=jax.ShapeDtypeStruct((B, 1), jnp.float32),
    )(ms, mc, cf, w1a, w1b, b1, prf, w2, b2)


def kernel(mesh_pos, mesh_norm, mesh_x, mesh_edge_index, mesh_batch,
           conn_x, conn_adj, params):
    p = params
    f32 = jnp.float32

    src = mesh_edge_index[0].astype(jnp.int32)
    dst = mesh_edge_index[1].astype(jnp.int32)

    pad = EPAD - E
    eid = lax.iota(jnp.int32, EPAD)
    srcp = jnp.concatenate([src, eid[:pad] % N])
    srcp = srcp.reshape(IDXROWS, CHUNK)
    dstp = jnp.concatenate([dst, jnp.full((pad,), -1, jnp.int32)])
    dummy_h = DUMMY + eid % (ACC - HALF)
    ld0 = jnp.where((dstp >= 0) & (dstp < HALF), dstp, dummy_h)
    ld1 = jnp.where(dstp >= HALF, dstp - HALF, dummy_h)
    ldst = jnp.stack([ld0, ld1]).reshape(2, IDXROWS, CHUNK)
    dummy_s = SDUMMY + eid % (SACC - N)
    didx2 = jnp.where(dstp >= 0, dstp, dummy_s).reshape(IDXROWS, CHUNK)

    zeros16 = jnp.zeros((ZROWS, 16), f32)
    ones16 = jnp.ones((CHUNK, 16), f32)
    degp = _sc_degree(ldst, zeros16, ones16)
    deg = degp.reshape(2, ACC, 16)[:, :HALF, :].reshape(N, 16)

    x0 = jnp.concatenate([mesh_pos, mesh_norm, mesh_x], axis=1)
    sums, csq, bmu = _stats(x0)

    zerosz = jnp.zeros((SZROWS, HW), f32)
    h_pre = x0
    bn_g, bn_b = p["bn0_g"], p["bn0_b"]
    for i in range(1, 5):
        c = h_pre.shape[1]
        y = _nms(h_pre, sums, csq, bmu, bn_g.reshape(1, c),
                 bn_b.reshape(1, c), p["gcnW%d" % i], deg)
        sp = _sc_scatter(y, srcp, didx2, zerosz)
        h_pre, sums, csq, bmu = _post(sp, y, deg,
                                      p["gcnb%d" % i].reshape(1, HID))
        bn_g, bn_b = p["bn%d_g" % i], p["bn%d_b" % i]

    batch2d = mesh_batch.astype(jnp.int32).reshape(N, 1)
    ms, mc = _pool(h_pre, sums, csq, bmu, bn_g.reshape(1, HID),
                   bn_b.reshape(1, HID), batch2d)

    ws = []
    for i in range(1, 4):
        ws += [p["relW%d" % i], p["relb%d" % i].reshape(1, HID),
               p["rootW%d" % i], p["pr%d" % i].reshape(1, 1)]
    h3 = _conn(conn_adj, conn_x, ws)
    cf = h3.reshape(B, NCONN * HID)

    out = _head(ms, mc, cf,
                p["lin1W"][:HID], p["lin1W"][HID:],
                p["lin1b"].reshape(1, HID), p["prF"].reshape(1, 1),
                p["lin2W"], p["lin2b"].reshape(1, 1))
    return out
